# Initial kernel scaffold; baseline (speedup 1.0000x reference)
import jax
import jax.numpy as jnp
from jax.experimental import pallas as pl


def kernel(text_feat, text_mask, img_feat, params):
    raise NotImplementedError("write your pallas kernel here")


# speedup vs baseline: 8.5127x; 8.5127x over previous
import functools
import math

import jax
import jax.numpy as jnp
from jax.experimental import pallas as pl
from jax.experimental.pallas import tpu as pltpu

C = 256
NQ = 10
NH = 8
DH = C // NH
FF = 512
NEG = float("-inf")


def _ln_rows(x, g, b):
    m = jnp.mean(x, axis=-1, keepdims=True)
    v = jnp.mean((x - m) ** 2, axis=-1, keepdims=True)
    return (x - m) / jnp.sqrt(v + 1e-5) * g + b


def _nn(a, b):
    return jax.lax.dot_general(a, b, (((1,), (0,)), ((), ())),
                               preferred_element_type=jnp.float32)


def _nt(a, b):
    return jax.lax.dot_general(a, b, (((1,), (1,)), ((), ())),
                               preferred_element_type=jnp.float32)


def _precompute_body(H, W, sa_Wo, sa_bo, sa_bv, n1g, n1b,
                     ca_Wv, ca_Wo, ca_bv, ca_bo,
                     posF_ref, M_ref, vec_ref):
    HW = H * W
    ci = jax.lax.broadcasted_iota(jnp.int32, (HW, C), 1)
    pi = jax.lax.broadcasted_iota(jnp.int32, (HW, C), 0)
    i = pi // W
    j = pi % W
    scale = 2.0 * math.pi
    yv = (i.astype(jnp.float32) + 1.0) / (H + 1e-6) * scale
    xv = (j.astype(jnp.float32) + 1.0) / (W + 1e-6) * scale
    k = (ci % (C // 2)) // 2
    tw = jnp.exp(k.astype(jnp.float32) * (2.0 / (C // 2)) * math.log(10000.0))
    val = jnp.where(ci < (C // 2), yv, xv) / tw
    posF_ref[...] = jnp.where(ci % 2 == 0, jnp.sin(val), jnp.cos(val))
    for h in range(NH):
        M_ref[h] = _nn(ca_Wv[:, h * DH:(h + 1) * DH],
                       ca_Wo[h * DH:(h + 1) * DH, :])
    c0 = _nn(sa_bv[...], sa_Wo[...]) + sa_bo[...]
    q1 = _ln_rows(c0, n1g[...], n1b[...])
    cb = _nn(ca_bv[...], ca_Wo[...]) + ca_bo[...]
    vec_ref[...] = jnp.concatenate(
        [q1, cb, jnp.zeros((6, C), jnp.float32)], axis=0)


def _main_body(L, HW, W,
               text_ref, img_ref, posF_ref, vec_ref,
               Wq_ref, bq_ref, Wk_ref, bk_ref, M_ref,
               fW1_ref, fb1_ref, fW2_ref, fb2_ref,
               n2g_ref, n2b_ref, n3g_ref, n3b_ref, png_ref, pnb_ref,
               mW1_ref, mb1_ref, mW2_ref, mb2_ref, mW3_ref, mb3_ref,
               x_ref, pts_ref, g_ref, attn_ref):
    tf = text_ref[0]
    rowi = jax.lax.broadcasted_iota(jnp.int32, (L, C), 0)
    cur = tf
    sels = []
    for _ in range(NQ):
        m = jnp.max(cur, axis=0, keepdims=True)
        idx = jnp.min(jnp.where(cur == m, rowi, L), axis=0, keepdims=True)
        sels.append(m)
        cur = jnp.where(rowi == idx, NEG, cur)
    sel = jnp.concatenate(sels, axis=0)

    q1 = vec_ref[0:1, :]
    cbias = vec_ref[1:2, :]
    qh = _nn(sel + q1, Wq_ref[...]) + bq_ref[...]

    kin = img_ref[0] + posF_ref[...]
    kh = _nn(kin, Wk_ref[...]) + bk_ref[...]
    s = jnp.concatenate(
        [_nt(qh[:, h * DH:(h + 1) * DH], kh[:, h * DH:(h + 1) * DH])
         for h in range(NH)], axis=0)
    s = s / math.sqrt(DH)
    p = jax.nn.softmax(s, axis=-1)

    am = jnp.mean(p.reshape(NH, NQ, HW), axis=0)
    g = jnp.max(am, axis=0, keepdims=True)
    g_ref[0] = g
    attn_ref[0] = am

    coli = jax.lax.broadcasted_iota(jnp.int32, (1, HW), 1)
    cur = g
    hots = []
    xs = []
    ys = []
    for _ in range(NQ):
        m = jnp.max(cur, axis=1, keepdims=True)
        idx = jnp.min(jnp.where(cur == m, coli, HW), axis=1, keepdims=True)
        hit = coli == idx
        hots.append(hit.astype(jnp.float32))
        cur = jnp.where(hit, NEG, cur)
        xs.append(((idx % W).astype(jnp.float32) + 0.5) / W)
        ys.append(((idx // W).astype(jnp.float32) + 0.5) / (HW // W))
    pts_ref[0] = jnp.concatenate(
        [jnp.concatenate(xs, axis=0), jnp.concatenate(ys, axis=0)], axis=1)

    a = jnp.concatenate([p] + hots + [jnp.zeros((6, HW), jnp.float32)],
                        axis=0)
    ctx = _nn(a, img_ref[0])

    ca = cbias
    for h in range(NH):
        ca = ca + _nn(ctx[h * NQ:(h + 1) * NQ, :], M_ref[h])
    q2 = _ln_rows(q1 + ca, n2g_ref[...], n2b_ref[...])
    ffn = _nn(jnp.maximum(_nn(q2, fW1_ref[...]) + fb1_ref[...], 0.0),
              fW2_ref[...]) + fb2_ref[...]
    q3 = _ln_rows(q2 + ffn, n3g_ref[...], n3b_ref[...])
    q4 = _ln_rows(q3, png_ref[...], pnb_ref[...])

    pos_feat = ctx[NH * NQ:NH * NQ + NQ, :]
    x = jnp.concatenate([q4, pos_feat], axis=1)
    x = jnp.maximum(_nn(x, mW1_ref[...]) + mb1_ref[...], 0.0)
    x = jnp.maximum(_nn(x, mW2_ref[...]) + mb2_ref[...], 0.0)
    x = _nn(x, mW3_ref[...]) + mb3_ref[...]
    x_ref[0] = x


def kernel(text_feat, text_mask, img_feat, params):
    del text_mask
    B, L, _ = text_feat.shape
    _, _, H, W = img_feat.shape
    HW = H * W
    img = img_feat.reshape(B, C, HW).transpose(0, 2, 1)
    p = params
    r = lambda v: v.reshape(1, -1)

    posF, M_all, vec = pl.pallas_call(
        functools.partial(_precompute_body, H, W),
        out_shape=[
            jax.ShapeDtypeStruct((HW, C), jnp.float32),
            jax.ShapeDtypeStruct((NH, C, C), jnp.float32),
            jax.ShapeDtypeStruct((8, C), jnp.float32),
        ],
    )(p['sa_Wo'], r(p['sa_bo']), r(p['sa_bv']), r(p['n1_g']), r(p['n1_b']),
      p['ca_Wv'], p['ca_Wo'], r(p['ca_bv']), r(p['ca_bo']))

    full = lambda shape: pl.BlockSpec(shape, lambda b: (0,) * len(shape))
    perb = lambda shape: pl.BlockSpec((1,) + shape,
                                      lambda b: (b,) + (0,) * len(shape))
    x, pts, g, am = pl.pallas_call(
        functools.partial(_main_body, L, HW, W),
        grid=(B,),
        in_specs=[
            perb((L, C)), perb((HW, C)), full((HW, C)), full((8, C)),
            full((C, C)), full((1, C)), full((C, C)), full((1, C)),
            full((NH, C, C)),
            full((C, FF)), full((1, FF)), full((FF, C)), full((1, C)),
            full((1, C)), full((1, C)), full((1, C)), full((1, C)),
            full((1, C)), full((1, C)),
            full((2 * C, C)), full((1, C)), full((C, C)), full((1, C)),
            full((C, C)), full((1, C)),
        ],
        out_specs=[perb((NQ, C)), perb((NQ, 2)), perb((1, HW)),
                   perb((NQ, HW))],
        out_shape=[
            jax.ShapeDtypeStruct((B, NQ, C), jnp.float32),
            jax.ShapeDtypeStruct((B, NQ, 2), jnp.float32),
            jax.ShapeDtypeStruct((B, 1, HW), jnp.float32),
            jax.ShapeDtypeStruct((B, NQ, HW), jnp.float32),
        ],
    )(text_feat, img, posF, vec,
      p['ca_Wq'], r(p['ca_bq']), p['ca_Wk'], r(p['ca_bk']), M_all,
      p['ffn_W1'], r(p['ffn_b1']), p['ffn_W2'], r(p['ffn_b2']),
      r(p['n2_g']), r(p['n2_b']), r(p['n3_g']), r(p['n3_b']),
      r(p['pn_g']), r(p['pn_b']),
      p['mlp_W1'], r(p['mlp_b1']), p['mlp_W2'], r(p['mlp_b2']),
      p['mlp_W3'], r(p['mlp_b3']))

    return (x, pts, g.reshape(B, H, W), am.reshape(B, NQ, H, W))


# speedup vs baseline: 18.3966x; 2.1611x over previous
import functools
import math

import jax
import jax.numpy as jnp
from jax.experimental import pallas as pl
from jax.experimental.pallas import tpu as pltpu

C = 256
NQ = 10
NH = 8
DH = C // NH
FF = 512
NEG = float("-inf")


def _ln_rows(x, g, b):
    m = jnp.mean(x, axis=-1, keepdims=True)
    v = jnp.mean((x - m) ** 2, axis=-1, keepdims=True)
    return (x - m) / jnp.sqrt(v + 1e-5) * g + b


def _nn(a, b):
    return jax.lax.dot_general(a, b, (((1,), (0,)), ((), ())),
                               preferred_element_type=jnp.float32)


def _nt(a, b):
    return jax.lax.dot_general(a, b, (((1,), (1,)), ((), ())),
                               preferred_element_type=jnp.float32)


def _precompute_body(H, W, sa_Wo, sa_bo, sa_bv, n1g, n1b,
                     ca_Wv, ca_Wo, ca_bv, ca_bo,
                     posF_ref, M_ref, vec_ref):
    HW = H * W
    ci = jax.lax.broadcasted_iota(jnp.int32, (HW, C), 1)
    pi = jax.lax.broadcasted_iota(jnp.int32, (HW, C), 0)
    i = pi // W
    j = pi % W
    scale = 2.0 * math.pi
    yv = (i.astype(jnp.float32) + 1.0) / (H + 1e-6) * scale
    xv = (j.astype(jnp.float32) + 1.0) / (W + 1e-6) * scale
    k = (ci % (C // 2)) // 2
    tw = jnp.exp(k.astype(jnp.float32) * (2.0 / (C // 2)) * math.log(10000.0))
    val = jnp.where(ci < (C // 2), yv, xv) / tw
    posF_ref[...] = jnp.where(ci % 2 == 0, jnp.sin(val), jnp.cos(val))
    for h in range(NH):
        M_ref[h] = _nn(ca_Wv[:, h * DH:(h + 1) * DH],
                       ca_Wo[h * DH:(h + 1) * DH, :])
    c0 = _nn(sa_bv[...], sa_Wo[...]) + sa_bo[...]
    q1 = _ln_rows(c0, n1g[...], n1b[...])
    cb = _nn(ca_bv[...], ca_Wo[...]) + ca_bo[...]
    vec_ref[...] = jnp.concatenate(
        [q1, cb, jnp.zeros((6, C), jnp.float32)], axis=0)


def _main_body(NB, L, HW, W,
               text_ref, img_ref, posF_ref, vec_ref,
               Wq_ref, bq_ref, Wk_ref, bk_ref, M_ref,
               fW1_ref, fb1_ref, fW2_ref, fb2_ref,
               n2g_ref, n2b_ref, n3g_ref, n3b_ref, png_ref, pnb_ref,
               mW1_ref, mb1_ref, mW2_ref, mb2_ref, mW3_ref, mb3_ref,
               x_ref, pts_ref, g_ref, attn_ref):
    tf = text_ref[...]
    rowi = jax.lax.broadcasted_iota(jnp.int32, (NB, L, C), 1)
    cur = tf
    sels = []
    for _ in range(NQ):
        m = jnp.max(cur, axis=1, keepdims=True)
        idx = jnp.min(jnp.where(cur == m, rowi, L), axis=1, keepdims=True)
        sels.append(m)
        cur = jnp.where(rowi == idx, NEG, cur)
    sel = jnp.concatenate(sels, axis=1)

    q1 = vec_ref[0:1, :]
    cbias = vec_ref[1:2, :]
    qh = _nn(sel.reshape(NB * NQ, C) + q1,
             Wq_ref[...]) + bq_ref[...]

    kin = (img_ref[...] + posF_ref[...][None]).reshape(NB * HW, C)
    kh = _nn(kin, Wk_ref[...]) + bk_ref[...]
    ss = []
    for i in range(NB):
        qh_i = qh[i * NQ:(i + 1) * NQ, :]
        kh_i = kh[i * HW:(i + 1) * HW, :]
        ss.extend(_nt(qh_i[:, h * DH:(h + 1) * DH],
                      kh_i[:, h * DH:(h + 1) * DH]) for h in range(NH))
    s = jnp.concatenate(ss, axis=0)
    s = s / math.sqrt(DH)
    p = jax.nn.softmax(s, axis=-1)

    am = jnp.mean(p.reshape(NB, NH, NQ, HW), axis=1)
    g = jnp.max(am, axis=1, keepdims=True)
    g_ref[...] = g
    attn_ref[...] = am

    coli = jax.lax.broadcasted_iota(jnp.int32, (NB, HW), 1)
    cur = g.reshape(NB, HW)
    hots = []
    xs = []
    ys = []
    for _ in range(NQ):
        m = jnp.max(cur, axis=1, keepdims=True)
        idx = jnp.min(jnp.where(cur == m, coli, HW), axis=1, keepdims=True)
        hit = coli == idx
        hots.append(hit.astype(jnp.float32)[:, None, :])
        cur = jnp.where(hit, NEG, cur)
        xs.append((((idx % W).astype(jnp.float32) + 0.5) / W)[:, None, :])
        ys.append((((idx // W).astype(jnp.float32) + 0.5)
                   / (HW // W))[:, None, :])
    pts_ref[...] = jnp.concatenate(
        [jnp.concatenate(xs, axis=1), jnp.concatenate(ys, axis=1)], axis=2)

    oh = jnp.concatenate(hots, axis=1)
    zpad = jnp.zeros((6, HW), jnp.float32)
    ctxs = [_nn(jnp.concatenate(
                [p[i * NH * NQ:(i + 1) * NH * NQ], oh[i], zpad], axis=0),
                img_ref[i]) for i in range(NB)]

    ca = cbias
    for h in range(NH):
        ch = jnp.concatenate([c[h * NQ:(h + 1) * NQ, :] for c in ctxs],
                             axis=0)
        ca = ca + _nn(ch, M_ref[h])
    q2 = _ln_rows(q1 + ca, n2g_ref[...], n2b_ref[...])
    ffn = _nn(jnp.maximum(_nn(q2, fW1_ref[...]) + fb1_ref[...], 0.0),
              fW2_ref[...]) + fb2_ref[...]
    q3 = _ln_rows(q2 + ffn, n3g_ref[...], n3b_ref[...])
    q4 = _ln_rows(q3, png_ref[...], pnb_ref[...])

    pos_feat = jnp.concatenate(
        [c[NH * NQ:NH * NQ + NQ, :] for c in ctxs], axis=0)
    x = jnp.concatenate([q4, pos_feat], axis=1)
    x = jnp.maximum(_nn(x, mW1_ref[...]) + mb1_ref[...], 0.0)
    x = jnp.maximum(_nn(x, mW2_ref[...]) + mb2_ref[...], 0.0)
    x = _nn(x, mW3_ref[...]) + mb3_ref[...]
    x_ref[...] = x.reshape(NB, NQ, C)


def kernel(text_feat, text_mask, img_feat, params):
    del text_mask
    B, L, _ = text_feat.shape
    _, _, H, W = img_feat.shape
    HW = H * W
    img = img_feat.reshape(B, C, HW).transpose(0, 2, 1)
    p = params
    r = lambda v: v.reshape(1, -1)

    posF, M_all, vec = pl.pallas_call(
        functools.partial(_precompute_body, H, W),
        out_shape=[
            jax.ShapeDtypeStruct((HW, C), jnp.float32),
            jax.ShapeDtypeStruct((NH, C, C), jnp.float32),
            jax.ShapeDtypeStruct((8, C), jnp.float32),
        ],
    )(p['sa_Wo'], r(p['sa_bo']), r(p['sa_bv']), r(p['n1_g']), r(p['n1_b']),
      p['ca_Wv'], p['ca_Wo'], r(p['ca_bv']), r(p['ca_bo']))

    NB = 4
    full = lambda shape: pl.BlockSpec(shape, lambda b: (0,) * len(shape))
    perb = lambda shape: pl.BlockSpec((NB,) + shape,
                                      lambda b: (b,) + (0,) * len(shape))
    x, pts, g, am = pl.pallas_call(
        functools.partial(_main_body, NB, L, HW, W),
        grid=(B // NB,),
        in_specs=[
            perb((L, C)), perb((HW, C)), full((HW, C)), full((8, C)),
            full((C, C)), full((1, C)), full((C, C)), full((1, C)),
            full((NH, C, C)),
            full((C, FF)), full((1, FF)), full((FF, C)), full((1, C)),
            full((1, C)), full((1, C)), full((1, C)), full((1, C)),
            full((1, C)), full((1, C)),
            full((2 * C, C)), full((1, C)), full((C, C)), full((1, C)),
            full((C, C)), full((1, C)),
        ],
        out_specs=[perb((NQ, C)), perb((NQ, 2)), perb((1, HW)),
                   perb((NQ, HW))],
        out_shape=[
            jax.ShapeDtypeStruct((B, NQ, C), jnp.float32),
            jax.ShapeDtypeStruct((B, NQ, 2), jnp.float32),
            jax.ShapeDtypeStruct((B, 1, HW), jnp.float32),
            jax.ShapeDtypeStruct((B, NQ, HW), jnp.float32),
        ],
    )(text_feat, img, posF, vec,
      p['ca_Wq'], r(p['ca_bq']), p['ca_Wk'], r(p['ca_bk']), M_all,
      p['ffn_W1'], r(p['ffn_b1']), p['ffn_W2'], r(p['ffn_b2']),
      r(p['n2_g']), r(p['n2_b']), r(p['n3_g']), r(p['n3_b']),
      r(p['pn_g']), r(p['pn_b']),
      p['mlp_W1'], r(p['mlp_b1']), p['mlp_W2'], r(p['mlp_b2']),
      p['mlp_W3'], r(p['mlp_b3']))

    return (x, pts, g.reshape(B, H, W), am.reshape(B, NQ, H, W))
